# Initial kernel scaffold; baseline (speedup 1.0000x reference)
import jax
import jax.numpy as jnp
from jax.experimental import pallas as pl


def kernel(pred_logits, pred_boxes, gt_boxes, gt_labels, head_idx):
    raise NotImplementedError("write your pallas kernel here")


# speedup vs baseline: 29.0383x; 29.0383x over previous
import functools

import jax
import jax.numpy as jnp
from jax import lax
from jax.experimental import pallas as pl

CENTER_RADIUS = 1.5
NUM_HEADS = 6
CLS_WEIGHT = 2.0
REG_WEIGHT = 0.25
IOU_WEIGHT = 0.25
ALPHA = 0.25
GAMMA = 2.0
EPS = 1e-12

NGT = 256
BIG_I = 1 << 30


def _pass1_body(pt_ref, gt_ref, cost_ref, ctop_ref, itop_ref, *, P):
    pt = pt_ref[0]
    g = gt_ref[0]

    def grow(i):
        return g[i:i + 1, :]

    px = pt[:, 0:1]
    py = pt[:, 1:2]
    pz = pt[:, 2:3]

    ib = ((px > grow(0)) & (px < grow(3)) &
          (py > grow(1)) & (py < grow(4)) &
          (pz > grow(2)) & (pz < grow(5)))
    ic = ((px > grow(6)) & (px < grow(9)) &
          (py > grow(7)) & (py < grow(10)) &
          (pz > grow(8)) & (pz < grow(11)))
    in_bc = ib & ic
    valid = (jnp.any(ib, axis=1, keepdims=True) |
             jnp.any(ic, axis=1, keepdims=True))

    labf = grow(27)
    cls_c = jnp.where(labf == 0.0, pt[:, 16:17], 0.0)
    for c in range(1, 10):
        cls_c = cls_c + jnp.where(labf == float(c), pt[:, 16 + c:17 + c], 0.0)

    reg = jnp.abs(pt[:, 0:1] - grow(12))
    for j in range(1, 8):
        reg = reg + jnp.abs(pt[:, j:j + 1] - grow(12 + j))
    reg_c = reg * REG_WEIGHT

    pdx = pt[:, 10:11]
    pdy = pt[:, 11:12]
    pdz = pt[:, 12:13]
    ltx = jnp.maximum(px - pdx * 0.5, grow(20))
    lty = jnp.maximum(py - pdy * 0.5, grow(21))
    ltz = jnp.maximum(pz - pdz * 0.5, grow(22))
    rbx = jnp.minimum(px + pdx * 0.5, grow(23))
    rby = jnp.minimum(py + pdy * 0.5, grow(24))
    rbz = jnp.minimum(pz + pdz * 0.5, grow(25))
    ix = jnp.maximum(rbx - ltx, 0.0)
    iy = jnp.maximum(rby - lty, 0.0)
    iz = jnp.maximum(rbz - ltz, 0.0)
    inter = (ix * iy) * iz
    va = (pdx * pdy) * pdz
    union = jnp.maximum(va + grow(26) - inter, 1e-6)
    iou = inter / union

    cost = cls_c + reg_c
    cost = cost + (-iou) * IOU_WEIGHT
    cost = cost + jnp.where(in_bc, 0.0, 100.0)
    cost = cost + jnp.where(valid, 0.0, 10000.0)

    col = lax.broadcasted_iota(jnp.int32, (P, NGT), 1)
    colmask = col < 200
    costm = jnp.where(colmask, cost, jnp.inf)
    cost_ref[0] = costm

    C = costm
    crows = []
    for j in range(5):
        m = jnp.min(C, axis=0, keepdims=True)
        crows.append(m)
        if j < 4:
            C = jnp.where(C == m, jnp.inf, C)
    ctop_ref[0] = jnp.concatenate(
        crows + [jnp.full((3, NGT), jnp.inf, jnp.float32)], axis=0)

    I = jnp.where(colmask, iou, -jnp.inf)
    irows = []
    for j in range(5):
        m = jnp.max(I, axis=0, keepdims=True)
        irows.append(jnp.maximum(m, 0.0))
        if j < 4:
            I = jnp.where(I == m, -jnp.inf, I)
    itop_ref[0] = jnp.concatenate(
        irows + [jnp.zeros((3, NGT), jnp.float32)], axis=0)


def _merge_body(ctop_ref, itop_ref, off_ref, thr_ref, *, R):
    ct = ctop_ref[0]
    it = itop_ref[0]
    off = off_ref[0]

    C = ct
    cv = []
    for j in range(5):
        m = jnp.min(C, axis=0, keepdims=True)
        cv.append(m)
        if j < 4:
            C = jnp.where(C == m, jnp.inf, C)

    I = it
    s = None
    for j in range(5):
        m = jnp.max(I, axis=0, keepdims=True)
        v = jnp.maximum(m, 0.0)
        s = v if s is None else s + v
        if j < 4:
            I = jnp.where(I == m, -jnp.inf, I)

    dk = jnp.clip((s - off).astype(jnp.int32), 1, 5)
    thr = jnp.where(dk == 1, cv[0],
                    jnp.where(dk == 2, cv[1],
                              jnp.where(dk == 3, cv[2],
                                        jnp.where(dk == 4, cv[3], cv[4]))))
    thr_ref[0] = thr


def _pass2_body(cost_ref, thr_ref, fg_ref, m_ref, *, P):
    C = cost_ref[0]
    thr = thr_ref[0]
    col = lax.broadcasted_iota(jnp.int32, (P, NGT), 1)
    colmask = col < 200
    match = (C <= thr) & colmask
    nm = jnp.sum(match.astype(jnp.int32), axis=1, keepdims=True)
    minv = jnp.min(C, axis=1, keepdims=True)
    amin = jnp.min(jnp.where(C == minv, col, BIG_I), axis=1, keepdims=True)
    fm = jnp.min(jnp.where(match, col, BIG_I), axis=1, keepdims=True)
    fg = nm > 0
    matched = jnp.where(fg, jnp.where(nm > 1, amin, fm), -1)
    fg_ref[0] = fg.astype(jnp.int32)
    m_ref[0] = matched


def _corners_minmax(boxes):
    signs = jnp.array([[1, 1, 1], [1, 1, -1], [1, -1, 1], [1, -1, -1],
                       [-1, 1, 1], [-1, 1, -1], [-1, -1, 1], [-1, -1, -1]],
                      dtype=jnp.float32) * 0.5
    corners = signs[None, :, :] * boxes[:, None, 3:6]
    ry = boxes[:, 6]
    c, s = jnp.cos(ry)[:, None], jnp.sin(ry)[:, None]
    x = corners[..., 0] * c - corners[..., 1] * s
    y = corners[..., 0] * s + corners[..., 1] * c
    pts = jnp.stack([x, y, corners[..., 2]], axis=-1) + boxes[:, None, 0:3]
    return jnp.min(pts, axis=1), jnp.max(pts, axis=1)


def kernel(pred_logits, pred_boxes, gt_boxes, gt_labels, head_idx):
    bs, n_p, _ = pred_logits.shape
    n_gt = gt_boxes.shape[1]
    P = 2000 if n_p % 2000 == 0 else n_p
    T = n_p // P

    p = jax.nn.sigmoid(pred_logits)
    neg = -jnp.log(1.0 - p + EPS) * (1.0 - ALPHA) * jnp.power(p, GAMMA)
    pos = -jnp.log(p + EPS) * ALPHA * jnp.power(1.0 - p, GAMMA)
    dfocal = (pos - neg) * CLS_WEIGHT
    pdims = jnp.exp(pred_boxes[..., 3:6])
    zero_p = jnp.zeros((bs, n_p, 3), jnp.float32)
    predtab = jnp.concatenate(
        [pred_boxes, pdims, zero_p, dfocal,
         jnp.zeros((bs, n_p, 6), jnp.float32)], axis=-1)

    gb = jnp.pad(gt_boxes, ((0, 0), (0, NGT - n_gt), (0, 0)))
    gbf = gb.reshape(bs * NGT, 7)
    mn, mx = _corners_minmax(gbf)
    mn = mn.reshape(bs, NGT, 3)
    mx = mx.reshape(bs, NGT, 3)
    gc = gb[..., 0:3]
    gd = gb[..., 3:6]
    lo = gc - CENTER_RADIUS * gd
    hi = gc + CENTER_RADIUS * gd
    rot = gb[..., 6:7]
    gnorm = jnp.concatenate(
        [gc, jnp.log(gd), jnp.sin(rot), jnp.cos(rot)], axis=-1)
    bmin = gc - gd * 0.5
    bmax = gc + gd * 0.5
    vb = (gd[..., 0:1] * gd[..., 1:2]) * gd[..., 2:3]
    labf = jnp.pad(gt_labels, ((0, 0), (0, NGT - n_gt))).astype(
        jnp.float32)[..., None]
    gttab = jnp.concatenate(
        [mn, mx, lo, hi, gnorm, bmin, bmax, vb, labf,
         jnp.zeros((bs, NGT, 4), jnp.float32)], axis=-1)
    gttab = gttab.transpose(0, 2, 1)

    off = 0.5 * (NUM_HEADS - head_idx)
    offs = jnp.broadcast_to(
        jnp.asarray(off, jnp.float32).reshape(1, 1, 1), (bs, 1, NGT))

    f32 = jnp.float32
    cost, ctop, itop = pl.pallas_call(
        functools.partial(_pass1_body, P=P),
        grid=(bs, T),
        in_specs=[
            pl.BlockSpec((1, P, 32), lambda b, t: (b, t, 0)),
            pl.BlockSpec((1, 32, NGT), lambda b, t: (b, 0, 0)),
        ],
        out_specs=[
            pl.BlockSpec((1, P, NGT), lambda b, t: (b, t, 0)),
            pl.BlockSpec((1, 8, NGT), lambda b, t: (b, t, 0)),
            pl.BlockSpec((1, 8, NGT), lambda b, t: (b, t, 0)),
        ],
        out_shape=[
            jax.ShapeDtypeStruct((bs, n_p, NGT), f32),
            jax.ShapeDtypeStruct((bs, T * 8, NGT), f32),
            jax.ShapeDtypeStruct((bs, T * 8, NGT), f32),
        ],
    )(predtab, gttab)

    thr = pl.pallas_call(
        functools.partial(_merge_body, R=T * 8),
        grid=(bs,),
        in_specs=[
            pl.BlockSpec((1, T * 8, NGT), lambda b: (b, 0, 0)),
            pl.BlockSpec((1, T * 8, NGT), lambda b: (b, 0, 0)),
            pl.BlockSpec((1, 1, NGT), lambda b: (b, 0, 0)),
        ],
        out_specs=pl.BlockSpec((1, 1, NGT), lambda b: (b, 0, 0)),
        out_shape=jax.ShapeDtypeStruct((bs, 1, NGT), f32),
    )(ctop, itop, offs)

    fgi, mt = pl.pallas_call(
        functools.partial(_pass2_body, P=P),
        grid=(bs, T),
        in_specs=[
            pl.BlockSpec((1, P, NGT), lambda b, t: (b, t, 0)),
            pl.BlockSpec((1, 1, NGT), lambda b, t: (b, 0, 0)),
        ],
        out_specs=[
            pl.BlockSpec((1, P, 1), lambda b, t: (b, t, 0)),
            pl.BlockSpec((1, P, 1), lambda b, t: (b, t, 0)),
        ],
        out_shape=[
            jax.ShapeDtypeStruct((bs, n_p, 1), jnp.int32),
            jax.ShapeDtypeStruct((bs, n_p, 1), jnp.int32),
        ],
    )(cost, thr)

    return fgi[..., 0] != 0, mt[..., 0]
